# baseline (speedup 1.0000x reference)
import jax
import jax.numpy as jnp
from jax import lax
from jax.experimental import pallas as pl
from jax.experimental.pallas import tpu as pltpu
from jax.experimental.pallas import tpu_sc as plsc

_L = 16
_DIM = 64
_TILE = 128


def _tec_body(idx_hbm, at_hbm, bt_hbm, c1_hbm, c2_hbm, out_hbm,
              ixv, abuf, bbuf, cb1, cb2, res, sem):
    @pl.when(jnp.logical_and(lax.axis_index("c") == 0,
                             lax.axis_index("s") == 0))
    def _():
        pltpu.sync_copy(idx_hbm, ixv)
        iv = ixv[...]
        xs = iv[0]
        ys = iv[8]
        xt = pl.multiple_of((xs // _TILE) * _TILE, _TILE)
        yt = pl.multiple_of((ys // _TILE) * _TILE, _TILE)
        d1 = pltpu.async_copy(at_hbm.at[:, pl.ds(xt, _TILE)], abuf, sem)
        d2 = pltpu.async_copy(bt_hbm.at[:, pl.ds(yt, _TILE)], bbuf, sem)
        d3 = pltpu.async_copy(c1_hbm.at[pl.ds(xt, _TILE)], cb1, sem)
        d4 = pltpu.async_copy(c2_hbm.at[pl.ds(yt, _TILE)], cb2, sem)
        d1.wait()
        d2.wait()
        d3.wait()
        d4.wait()
        lx = xs - xt
        ly = ys - yt
        bx = jnp.minimum(lx, _TILE - _L)
        by = jnp.minimum(ly, _TILE - _L)
        jxv = jnp.full((_L,), lx - bx, jnp.int32)
        jyv = jnp.full((_L,), ly - by, jnp.int32)

        def step(d, acc):
            va = abuf[d, pl.ds(bx, _L)]
            vb = bbuf[d, pl.ds(by, _L)]
            aa = va.at[jxv].get(mode="promise_in_bounds")
            return acc + aa * vb

        acc = lax.fori_loop(0, _DIM, step, jnp.zeros((_L,), jnp.float32))
        dot_all = acc.at[jyv].get(mode="promise_in_bounds")
        b1 = cb1[pl.ds(bx, _L)].at[jxv].get(mode="promise_in_bounds")
        b2 = cb2[pl.ds(by, _L)].at[jyv].get(mode="promise_in_bounds")
        res[...] = dot_all + b1 + b2
        pltpu.sync_copy(res.at[pl.ds(0, 1)], out_hbm)


def kernel(x, y, A, B, c1, c2):
    idx = jnp.concatenate([
        jnp.full((8,), x, dtype=jnp.int32),
        jnp.full((8,), y, dtype=jnp.int32),
    ])
    run = pl.kernel(
        _tec_body,
        mesh=plsc.VectorSubcoreMesh(core_axis_name="c", subcore_axis_name="s",
                                    num_cores=1, num_subcores=1),
        out_type=jax.ShapeDtypeStruct((1,), jnp.float32),
        scratch_types=[
            pltpu.VMEM((_L,), jnp.int32),
            pltpu.VMEM((_DIM, _TILE), jnp.float32),
            pltpu.VMEM((_DIM, _TILE), jnp.float32),
            pltpu.VMEM((_TILE,), jnp.float32),
            pltpu.VMEM((_TILE,), jnp.float32),
            pltpu.VMEM((_L,), jnp.float32),
            pltpu.SemaphoreType.DMA,
        ],
    )
    return run(idx, A.T, B.T, jnp.reshape(c1, (-1,)), jnp.reshape(c2, (-1,)))


# speedup vs baseline: 1.0054x; 1.0054x over previous
import jax
import jax.numpy as jnp
from jax import lax
from jax.experimental import pallas as pl
from jax.experimental.pallas import tpu as pltpu
from jax.experimental.pallas import tpu_sc as plsc

_L = 16
_DIM = 64
_TILE = 128


def _tec_body(idx_hbm, at_hbm, bt_hbm, c1_hbm, c2_hbm, out_hbm,
              ixv, abuf, bbuf, cb1, cb2, res, sem):
    @pl.when(jnp.logical_and(lax.axis_index("c") == 0,
                             lax.axis_index("s") == 0))
    def _():
        pltpu.sync_copy(idx_hbm, ixv)
        iv = ixv[...]
        xs = iv[0]
        ys = iv[8]
        xt = pl.multiple_of((xs // _TILE) * _TILE, _TILE)
        yt = pl.multiple_of((ys // _TILE) * _TILE, _TILE)
        d1 = pltpu.async_copy(at_hbm.at[:, pl.ds(xt, _TILE)], abuf, sem)
        d2 = pltpu.async_copy(bt_hbm.at[:, pl.ds(yt, _TILE)], bbuf, sem)
        d3 = pltpu.async_copy(c1_hbm.at[pl.ds(xt, _TILE)], cb1, sem)
        d4 = pltpu.async_copy(c2_hbm.at[pl.ds(yt, _TILE)], cb2, sem)
        d1.wait()
        d2.wait()
        d3.wait()
        d4.wait()
        lx = xs - xt
        ly = ys - yt
        bx = jnp.minimum(lx, _TILE - _L)
        by = jnp.minimum(ly, _TILE - _L)
        jxv = jnp.full((_L,), lx - bx, jnp.int32)
        jyv = jnp.full((_L,), ly - by, jnp.int32)

        acc = jnp.zeros((_L,), jnp.float32)
        for d in range(_DIM):
            va = abuf[d, pl.ds(bx, _L)]
            vb = bbuf[d, pl.ds(by, _L)]
            aa = va.at[jxv].get(mode="promise_in_bounds")
            acc = acc + aa * vb
        dot_all = acc.at[jyv].get(mode="promise_in_bounds")
        b1 = cb1[pl.ds(bx, _L)].at[jxv].get(mode="promise_in_bounds")
        b2 = cb2[pl.ds(by, _L)].at[jyv].get(mode="promise_in_bounds")
        res[...] = dot_all + b1 + b2
        pltpu.sync_copy(res.at[pl.ds(0, 1)], out_hbm)


def kernel(x, y, A, B, c1, c2):
    idx = jnp.concatenate([
        jnp.full((8,), x, dtype=jnp.int32),
        jnp.full((8,), y, dtype=jnp.int32),
    ])
    run = pl.kernel(
        _tec_body,
        mesh=plsc.VectorSubcoreMesh(core_axis_name="c", subcore_axis_name="s",
                                    num_cores=1, num_subcores=1),
        out_type=jax.ShapeDtypeStruct((1,), jnp.float32),
        scratch_types=[
            pltpu.VMEM((_L,), jnp.int32),
            pltpu.VMEM((_DIM, _TILE), jnp.float32),
            pltpu.VMEM((_DIM, _TILE), jnp.float32),
            pltpu.VMEM((_TILE,), jnp.float32),
            pltpu.VMEM((_TILE,), jnp.float32),
            pltpu.VMEM((_L,), jnp.float32),
            pltpu.SemaphoreType.DMA,
        ],
    )
    return run(idx, A.T, B.T, jnp.reshape(c1, (-1,)), jnp.reshape(c2, (-1,)))


# speedup vs baseline: 1.0092x; 1.0038x over previous
import jax
import jax.numpy as jnp
from jax import lax
from jax.experimental import pallas as pl
from jax.experimental.pallas import tpu as pltpu
from jax.experimental.pallas import tpu_sc as plsc

_L = 16
_DIM = 64
_TILE = 128


def _tec_body(idx_hbm, at_hbm, bt_hbm, c1_hbm, c2_hbm, out_hbm,
              ixv, abuf, bbuf, cb1, cb2, res, sem):
    @pl.when(jnp.logical_and(lax.axis_index("c") == 0,
                             lax.axis_index("s") == 0))
    def _():
        pltpu.sync_copy(idx_hbm, ixv)
        iv = ixv[...]
        xs = iv[0]
        ys = iv[8]
        xt = pl.multiple_of((xs // _TILE) * _TILE, _TILE)
        yt = pl.multiple_of((ys // _TILE) * _TILE, _TILE)
        d1 = pltpu.async_copy(at_hbm.at[:, pl.ds(xt, _TILE)], abuf, sem)
        d2 = pltpu.async_copy(bt_hbm.at[:, pl.ds(yt, _TILE)], bbuf, sem)
        d3 = pltpu.async_copy(c1_hbm.at[pl.ds(xt, _TILE)], cb1, sem)
        d4 = pltpu.async_copy(c2_hbm.at[pl.ds(yt, _TILE)], cb2, sem)
        d1.wait()
        d2.wait()
        d3.wait()
        d4.wait()
        lx = xs - xt
        ly = ys - yt
        bx = jnp.minimum(lx, _TILE - _L)
        by = jnp.minimum(ly, _TILE - _L)
        jxv = jnp.full((_L,), lx - bx, jnp.int32)
        jyv = jnp.full((_L,), ly - by, jnp.int32)

        acc = jnp.zeros((_L,), jnp.float32)
        for d in range(_DIM):
            va = abuf[d, pl.ds(bx, _L)]
            vb = bbuf[d, pl.ds(by, _L)]
            aa = va.at[jxv].get(mode="promise_in_bounds")
            acc = acc + aa * vb
        dot_all = acc.at[jyv].get(mode="promise_in_bounds")
        b1 = cb1[pl.ds(bx, _L)].at[jxv].get(mode="promise_in_bounds")
        b2 = cb2[pl.ds(by, _L)].at[jyv].get(mode="promise_in_bounds")
        res[...] = dot_all + b1 + b2
        pltpu.sync_copy(res.at[pl.ds(0, 1)], out_hbm)


def kernel(x, y, A, B, c1, c2):
    xi = jnp.asarray(x, jnp.int32)
    yi = jnp.asarray(y, jnp.int32)
    idx = jnp.where(jnp.arange(16) < 8, xi, yi)
    run = pl.kernel(
        _tec_body,
        mesh=plsc.VectorSubcoreMesh(core_axis_name="c", subcore_axis_name="s",
                                    num_cores=1, num_subcores=1),
        out_type=jax.ShapeDtypeStruct((1,), jnp.float32),
        scratch_types=[
            pltpu.VMEM((_L,), jnp.int32),
            pltpu.VMEM((_DIM, _TILE), jnp.float32),
            pltpu.VMEM((_DIM, _TILE), jnp.float32),
            pltpu.VMEM((_TILE,), jnp.float32),
            pltpu.VMEM((_TILE,), jnp.float32),
            pltpu.VMEM((_L,), jnp.float32),
            pltpu.SemaphoreType.DMA,
        ],
    )
    return run(idx, A.T, B.T, jnp.reshape(c1, (-1,)), jnp.reshape(c2, (-1,)))
